# baseline (device time: 3494 ns/iter reference)
import jax
import jax.numpy as jnp
from jax import lax
from jax.experimental import pallas as pl
from jax.experimental.pallas import tpu as pltpu

N_X = 2
GRID = 8


def kernel(x):
    m, n = x.shape
    m_blk = m // GRID
    inv_rows = 1.0 / (N_X * m)

    def body(x_ref, out_ref, acc_ref):
        i = pl.program_id(0)
        chunk = jnp.sum(x_ref[:, :], axis=0, keepdims=True)

        @pl.when(i == 0)
        def _():
            acc_ref[:, :] = chunk

        @pl.when(i > 0)
        def _():
            acc_ref[:, :] = acc_ref[:, :] + chunk

        @pl.when(i == GRID - 1)
        def _():
            out_ref[:, :] = acc_ref[:, :] * inv_rows

    return pl.pallas_call(
        body,
        grid=(GRID,),
        out_shape=jax.ShapeDtypeStruct((1, n), jnp.float32),
        in_specs=[pl.BlockSpec((m_blk, n), lambda i: (i, 0))],
        out_specs=pl.BlockSpec((1, n), lambda i: (0, 0)),
        scratch_shapes=[pltpu.VMEM((1, n), jnp.float32)],
    )(x)
